# baseline (speedup 1.0000x reference)
import functools

import jax
import jax.numpy as jnp
from jax import lax
from jax.experimental import pallas as pl
from jax.experimental.pallas import tpu as pltpu
from jax.experimental.pallas import tpu_sc as plsc

DIM = 16
NROWS = 1_000_000
BATCH = 16384
NUM_CORES = 2
NUM_SUBCORES = 16
NW = NUM_CORES * NUM_SUBCORES
BPW = BATCH // NW
CHUNK = 128
NCHUNK = BPW // CHUNK
NBLK = BPW // 16

_mesh = plsc.VectorSubcoreMesh(
    core_axis_name="c", subcore_axis_name="s",
    num_cores=NUM_CORES, num_subcores=NUM_SUBCORES)


@functools.partial(
    pl.kernel,
    out_type=jax.ShapeDtypeStruct((BATCH,), jnp.float32),
    mesh=_mesh,
    scratch_types=[
        pltpu.VMEM((BPW,), jnp.int32),
        pltpu.VMEM((BPW,), jnp.int32),
        pltpu.VMEM((DIM, BPW), jnp.float32),
        pltpu.VMEM((DIM, BPW), jnp.float32),
        pltpu.VMEM((BPW,), jnp.float32),
        pltpu.SemaphoreType.DMA,
    ],
    compiler_params=pltpu.CompilerParams(needs_layout_passes=False),
)
def _cosine_sc(e_id_hbm, o_id_hbm, ef_hbm, of_hbm, out_hbm,
               eidx_v, oidx_v, ebuf_v, obuf_v, out_v, sem):
    wid = lax.axis_index("s") * NUM_CORES + lax.axis_index("c")
    base = wid * BPW

    pltpu.sync_copy(e_id_hbm.at[pl.ds(base, BPW)], eidx_v)
    pltpu.sync_copy(o_id_hbm.at[pl.ds(base, BPW)], oidx_v)

    for table_hbm, idx_v, dst_v in (
        (ef_hbm, eidx_v, ebuf_v),
        (of_hbm, oidx_v, obuf_v),
    ):
        for d in range(DIM):
            row = table_hbm.at[pl.ds(d * NROWS, NROWS)]
            for j in range(NCHUNK):
                idx = idx_v.at[pl.ds(j * CHUNK, CHUNK)]
                dst = dst_v.at[d, pl.ds(j * CHUNK, CHUNK)]
                pltpu.async_copy(row.at[idx], dst, sem)

    for d in range(DIM):
        pltpu.make_async_copy(
            ef_hbm.at[pl.ds(0, BPW)], ebuf_v.at[d], sem).wait()
        pltpu.make_async_copy(
            of_hbm.at[pl.ds(0, BPW)], obuf_v.at[d], sem).wait()

    def block_body(b, carry):
        sl = pl.ds(b * 16, 16)
        dot = jnp.zeros((16,), jnp.float32)
        ee = jnp.zeros((16,), jnp.float32)
        oo = jnp.zeros((16,), jnp.float32)
        for d in range(DIM):
            ge = ebuf_v[d, sl]
            go = obuf_v[d, sl]
            dot = dot + ge * go
            ee = ee + ge * ge
            oo = oo + go * go
        x = ee * oo
        i = plsc.bitcast(x, jnp.int32)
        i = jnp.int32(0x5F3759DF) - lax.shift_right_logical(i, 1)
        y = plsc.bitcast(i, jnp.float32)
        hx = x * jnp.float32(0.5)
        for _ in range(3):
            y = y * (jnp.float32(1.5) - hx * y * y)
        out_v[sl] = dot * y
        return carry

    lax.fori_loop(0, NBLK, block_body, jnp.int32(0))

    pltpu.sync_copy(out_v, out_hbm.at[pl.ds(base, BPW)])


def kernel(e_id, o_id, mentees, mentors):
    ef = mentees.T.reshape(-1)
    of = mentors.T.reshape(-1)
    return _cosine_sc(e_id, o_id, ef, of)


# speedup vs baseline: 1.4914x; 1.4914x over previous
import functools

import jax
import jax.numpy as jnp
from jax import lax
from jax.experimental import pallas as pl
from jax.experimental.pallas import tpu as pltpu
from jax.experimental.pallas import tpu_sc as plsc

DIM = 16
NROWS = 1_000_000
BATCH = 16384
NUM_CORES = 2
NUM_SUBCORES = 16
NW = NUM_CORES * NUM_SUBCORES
BPW = BATCH // NW
CHUNK = 128
NCHUNK = BPW // CHUNK
NBLK = BPW // 16

_mesh = plsc.VectorSubcoreMesh(
    core_axis_name="c", subcore_axis_name="s",
    num_cores=NUM_CORES, num_subcores=NUM_SUBCORES)


@functools.partial(
    pl.kernel,
    out_type=jax.ShapeDtypeStruct((BATCH,), jnp.float32),
    mesh=_mesh,
    scratch_types=[
        pltpu.VMEM((BPW,), jnp.int32),
        pltpu.VMEM((BPW,), jnp.int32),
        pltpu.VMEM((DIM, BPW), jnp.float32),
        pltpu.VMEM((DIM, BPW), jnp.float32),
        pltpu.VMEM((BPW,), jnp.float32),
        pltpu.SemaphoreType.DMA,
    ],
    compiler_params=pltpu.CompilerParams(needs_layout_passes=False),
)
def _cosine_sc(e_id_hbm, o_id_hbm, ef_hbm, of_hbm, out_hbm,
               eidx_v, oidx_v, ebuf_v, obuf_v, out_v, sem):
    wid = lax.axis_index("s") * NUM_CORES + lax.axis_index("c")
    base = wid * BPW

    pltpu.sync_copy(e_id_hbm.at[pl.ds(base, BPW)], eidx_v)
    pltpu.sync_copy(o_id_hbm.at[pl.ds(base, BPW)], oidx_v)

    for table_hbm, idx_v, dst_v in (
        (ef_hbm, eidx_v, ebuf_v),
        (of_hbm, oidx_v, obuf_v),
    ):
        for d in range(DIM):
            row = table_hbm.at[pl.ds(d * NROWS, NROWS)]
            for j in range(NCHUNK):
                idx = idx_v.at[pl.ds(j * CHUNK, CHUNK)]
                dst = dst_v.at[d, pl.ds(j * CHUNK, CHUNK)]
                pltpu.async_copy(row.at[idx], dst, sem)

    for d in range(DIM):
        pltpu.make_async_copy(
            ef_hbm.at[pl.ds(0, BPW)], ebuf_v.at[d], sem).wait()
        pltpu.make_async_copy(
            of_hbm.at[pl.ds(0, BPW)], obuf_v.at[d], sem).wait()

    def block_body(b, carry):
        sl = pl.ds(b * 16, 16)
        dot = jnp.zeros((16,), jnp.float32)
        ee = jnp.zeros((16,), jnp.float32)
        oo = jnp.zeros((16,), jnp.float32)
        for d in range(DIM):
            ge = ebuf_v[d, sl]
            go = obuf_v[d, sl]
            dot = dot + ge * go
            ee = ee + ge * ge
            oo = oo + go * go
        x = ee * oo
        i = plsc.bitcast(x, jnp.int32)
        i = jnp.int32(0x5F3759DF) - lax.shift_right_logical(i, 1)
        y = plsc.bitcast(i, jnp.float32)
        hx = x * jnp.float32(0.5)
        for _ in range(3):
            y = y * (jnp.float32(1.5) - hx * y * y)
        out_v[sl] = dot * y
        return carry

    lax.fori_loop(0, NBLK, block_body, jnp.int32(0))

    pltpu.sync_copy(out_v, out_hbm.at[pl.ds(base, BPW)])


def kernel(e_id, o_id, mentees, mentors):
    ef = jnp.concatenate([mentees[:, d] for d in range(DIM)])
    of = jnp.concatenate([mentors[:, d] for d in range(DIM)])
    return _cosine_sc(e_id, o_id, ef, of)


# speedup vs baseline: 15.7858x; 10.5849x over previous
import functools

import jax
import jax.numpy as jnp
from jax import lax
from jax.experimental import pallas as pl
from jax.experimental.pallas import tpu as pltpu
from jax.experimental.pallas import tpu_sc as plsc

DIM = 16
BATCH = 16384
NUM_CORES = 2
NUM_SUBCORES = 16
NW = NUM_CORES * NUM_SUBCORES
BPW = BATCH // NW
NGRP = BPW // 16
PANEL = 128
NROWS = 1_000_000
LAST_PANEL = (NROWS // PANEL - 1) * PANEL
TAIL = PANEL
TAIL_START = NROWS - TAIL

_mesh = plsc.VectorSubcoreMesh(
    core_axis_name="c", subcore_axis_name="s",
    num_cores=NUM_CORES, num_subcores=NUM_SUBCORES)


@functools.partial(
    pl.kernel,
    out_type=jax.ShapeDtypeStruct((BATCH,), jnp.float32),
    mesh=_mesh,
    scratch_types=[
        pltpu.VMEM((BPW,), jnp.int32),
        pltpu.VMEM((BPW,), jnp.int32),
        pltpu.VMEM((DIM, TAIL), jnp.float32),
        pltpu.VMEM((DIM, TAIL), jnp.float32),
        pltpu.VMEM((16, DIM, PANEL), jnp.float32),
        pltpu.VMEM((16, DIM, PANEL), jnp.float32),
        pltpu.VMEM((DIM, BPW), jnp.float32),
        pltpu.VMEM((DIM, BPW), jnp.float32),
        pltpu.VMEM((BPW,), jnp.float32),
        pltpu.SemaphoreType.DMA,
        pltpu.SemaphoreType.DMA,
    ],
    compiler_params=pltpu.CompilerParams(
        needs_layout_passes=False, use_tc_tiling_on_sc=True),
)
def _cosine_sc(e_id_hbm, o_id_hbm, mentees_t_hbm, mentors_t_hbm,
               tail_e_hbm, tail_o_hbm, out_hbm,
               eidx_v, oidx_v, taile_v, tailo_v, epan_v, opan_v,
               ebuf_v, obuf_v, out_v, sem_e, sem_o):
    wid = lax.axis_index("s") * NUM_CORES + lax.axis_index("c")
    base = wid * BPW

    pltpu.sync_copy(e_id_hbm.at[pl.ds(base, BPW)], eidx_v)
    pltpu.sync_copy(o_id_hbm.at[pl.ds(base, BPW)], oidx_v)
    pltpu.sync_copy(tail_e_hbm, taile_v)
    pltpu.sync_copy(tail_o_hbm, tailo_v)

    lanes = lax.iota(jnp.int32, 16)

    def panel_start(sid):
        p = lax.min(lax.shift_right_logical(sid, 7) * PANEL,
                    jnp.int32(LAST_PANEL))
        return pl.multiple_of(p, PANEL)

    def pick(sid, pan_ref, tail_v):
        lane = lax.min(sid - panel_start(sid), jnp.int32(PANEL - 1))
        col = plsc.load_gather(pan_ref, [lanes, jnp.full((16,), lane,
                                                         jnp.int32)])
        lane_t = lax.max(sid - jnp.int32(TAIL_START), jnp.int32(0))
        col_t = plsc.load_gather(tail_v, [lanes, jnp.full((16,), lane_t,
                                                          jnp.int32)])
        return jnp.where(sid >= TAIL_START, col_t, col)

    def group_body(g, carry):
        evec = eidx_v[pl.ds(g * 16, 16)]
        ovec = oidx_v[pl.ds(g * 16, 16)]
        for l in range(16):
            pltpu.async_copy(
                mentees_t_hbm.at[:, pl.ds(panel_start(evec[l]), PANEL)],
                epan_v.at[l], sem_e)
            pltpu.async_copy(
                mentors_t_hbm.at[:, pl.ds(panel_start(ovec[l]), PANEL)],
                opan_v.at[l], sem_o)
        for l in range(16):
            pltpu.make_async_copy(
                mentees_t_hbm.at[:, pl.ds(0, PANEL)], epan_v.at[l],
                sem_e).wait()
            pltpu.make_async_copy(
                mentors_t_hbm.at[:, pl.ds(0, PANEL)], opan_v.at[l],
                sem_o).wait()
        for l in range(16):
            col_e = pick(evec[l], epan_v.at[l], taile_v)
            col_o = pick(ovec[l], opan_v.at[l], tailo_v)
            dst = jnp.full((16,), g * 16 + l, jnp.int32)
            plsc.store_scatter(ebuf_v, [lanes, dst], col_e)
            plsc.store_scatter(obuf_v, [lanes, dst], col_o)
        return carry

    lax.fori_loop(0, NGRP, group_body, jnp.int32(0))

    def block_body(b, carry):
        sl = pl.ds(b * 16, 16)
        dot = jnp.zeros((16,), jnp.float32)
        ee = jnp.zeros((16,), jnp.float32)
        oo = jnp.zeros((16,), jnp.float32)
        for d in range(DIM):
            ge = ebuf_v[d, sl]
            go = obuf_v[d, sl]
            dot = dot + ge * go
            ee = ee + ge * ge
            oo = oo + go * go
        x = ee * oo
        i = plsc.bitcast(x, jnp.int32)
        i = jnp.int32(0x5F3759DF) - lax.shift_right_logical(i, 1)
        y = plsc.bitcast(i, jnp.float32)
        hx = x * jnp.float32(0.5)
        for _ in range(3):
            y = y * (jnp.float32(1.5) - hx * y * y)
        out_v[sl] = dot * y
        return carry

    lax.fori_loop(0, NGRP, block_body, jnp.int32(0))

    pltpu.sync_copy(out_v, out_hbm.at[pl.ds(base, BPW)])


def kernel(e_id, o_id, mentees, mentors):
    tail_e = mentees[TAIL_START:, :].T
    tail_o = mentors[TAIL_START:, :].T
    return _cosine_sc(e_id, o_id, mentees.T, mentors.T, tail_e, tail_o)


# speedup vs baseline: 18.8596x; 1.1947x over previous
import functools

import jax
import jax.numpy as jnp
from jax import lax
from jax.experimental import pallas as pl
from jax.experimental.pallas import tpu as pltpu
from jax.experimental.pallas import tpu_sc as plsc

DIM = 16
BATCH = 16384
NUM_CORES = 2
NUM_SUBCORES = 16
NW = NUM_CORES * NUM_SUBCORES
BPW = BATCH // NW
NGRP = BPW // 16
PANEL = 128
NROWS = 1_000_000
LAST_PANEL = (NROWS // PANEL - 1) * PANEL
TAIL = PANEL
TAIL_START = NROWS - TAIL

_mesh = plsc.VectorSubcoreMesh(
    core_axis_name="c", subcore_axis_name="s",
    num_cores=NUM_CORES, num_subcores=NUM_SUBCORES)


@functools.partial(
    pl.kernel,
    out_type=jax.ShapeDtypeStruct((BATCH,), jnp.float32),
    mesh=_mesh,
    scratch_types=[
        pltpu.VMEM((BPW,), jnp.int32),
        pltpu.VMEM((BPW,), jnp.int32),
        pltpu.VMEM((DIM, TAIL), jnp.float32),
        pltpu.VMEM((DIM, TAIL), jnp.float32),
        pltpu.VMEM((16, DIM, PANEL), jnp.float32),
        pltpu.VMEM((16, DIM, PANEL), jnp.float32),
        pltpu.VMEM((DIM, BPW), jnp.float32),
        pltpu.VMEM((DIM, BPW), jnp.float32),
        pltpu.VMEM((BPW,), jnp.float32),
        pltpu.SemaphoreType.DMA,
        pltpu.SemaphoreType.DMA,
        pltpu.SemaphoreType.DMA,
        pltpu.SemaphoreType.DMA,
    ],
    compiler_params=pltpu.CompilerParams(
        needs_layout_passes=False, use_tc_tiling_on_sc=True),
)
def _cosine_sc(e_id_hbm, o_id_hbm, mentees_t_hbm, mentors_t_hbm,
               tail_e_hbm, tail_o_hbm, out_hbm,
               eidx_v, oidx_v, taile_v, tailo_v, epan_v, opan_v,
               ebuf_v, obuf_v, out_v, sem_ea, sem_eb, sem_oa, sem_ob):
    wid = lax.axis_index("s") * NUM_CORES + lax.axis_index("c")
    base = wid * BPW

    pltpu.sync_copy(e_id_hbm.at[pl.ds(base, BPW)], eidx_v)
    pltpu.sync_copy(o_id_hbm.at[pl.ds(base, BPW)], oidx_v)
    pltpu.sync_copy(tail_e_hbm, taile_v)
    pltpu.sync_copy(tail_o_hbm, tailo_v)

    lanes = lax.iota(jnp.int32, 16)

    def panel_start(sid):
        p = lax.min(lax.shift_right_logical(sid, 7) * PANEL,
                    jnp.int32(LAST_PANEL))
        return pl.multiple_of(p, PANEL)

    def pick(sid, pan_ref, tail_v):
        lane = lax.min(sid - panel_start(sid), jnp.int32(PANEL - 1))
        col = plsc.load_gather(pan_ref, [lanes, jnp.full((16,), lane,
                                                         jnp.int32)])
        lane_t = lax.max(sid - jnp.int32(TAIL_START), jnp.int32(0))
        col_t = plsc.load_gather(tail_v, [lanes, jnp.full((16,), lane_t,
                                                          jnp.int32)])
        return jnp.where(sid >= TAIL_START, col_t, col)

    half_sems = ((sem_ea, sem_oa), (sem_eb, sem_ob))

    def issue_half(evec, ovec, half):
        se, so = half_sems[half]
        for l in range(8):
            k = half * 8 + l
            pltpu.async_copy(
                mentees_t_hbm.at[:, pl.ds(panel_start(evec[k]), PANEL)],
                epan_v.at[k], se)
            pltpu.async_copy(
                mentors_t_hbm.at[:, pl.ds(panel_start(ovec[k]), PANEL)],
                opan_v.at[k], so)

    def drain_half(half):
        se, so = half_sems[half]
        for l in range(8):
            k = half * 8 + l
            pltpu.make_async_copy(
                mentees_t_hbm.at[:, pl.ds(0, PANEL)], epan_v.at[k],
                se).wait()
            pltpu.make_async_copy(
                mentors_t_hbm.at[:, pl.ds(0, PANEL)], opan_v.at[k],
                so).wait()

    def extract_half(g, evec, ovec, half):
        for l in range(8):
            k = half * 8 + l
            col_e = pick(evec[k], epan_v.at[k], taile_v)
            col_o = pick(ovec[k], opan_v.at[k], tailo_v)
            dst = jnp.full((16,), g * 16 + k, jnp.int32)
            plsc.store_scatter(ebuf_v, [lanes, dst], col_e)
            plsc.store_scatter(obuf_v, [lanes, dst], col_o)

    ev0 = eidx_v[pl.ds(0, 16)]
    ov0 = oidx_v[pl.ds(0, 16)]
    issue_half(ev0, ov0, 0)
    issue_half(ev0, ov0, 1)

    def group_body(g, carry):
        evec = eidx_v[pl.ds(g * 16, 16)]
        ovec = oidx_v[pl.ds(g * 16, 16)]
        evn = eidx_v[pl.ds((g + 1) * 16, 16)]
        ovn = oidx_v[pl.ds((g + 1) * 16, 16)]
        drain_half(0)
        extract_half(g, evec, ovec, 0)
        issue_half(evn, ovn, 0)
        drain_half(1)
        extract_half(g, evec, ovec, 1)
        issue_half(evn, ovn, 1)
        return carry

    lax.fori_loop(0, NGRP - 1, group_body, jnp.int32(0))

    evl = eidx_v[pl.ds((NGRP - 1) * 16, 16)]
    ovl = oidx_v[pl.ds((NGRP - 1) * 16, 16)]
    drain_half(0)
    extract_half(NGRP - 1, evl, ovl, 0)
    drain_half(1)
    extract_half(NGRP - 1, evl, ovl, 1)

    def block_body(b, carry):
        sl = pl.ds(b * 16, 16)
        dot = jnp.zeros((16,), jnp.float32)
        ee = jnp.zeros((16,), jnp.float32)
        oo = jnp.zeros((16,), jnp.float32)
        for d in range(DIM):
            ge = ebuf_v[d, sl]
            go = obuf_v[d, sl]
            dot = dot + ge * go
            ee = ee + ge * ge
            oo = oo + go * go
        x = ee * oo
        i = plsc.bitcast(x, jnp.int32)
        i = jnp.int32(0x5F3759DF) - lax.shift_right_logical(i, 1)
        y = plsc.bitcast(i, jnp.float32)
        hx = x * jnp.float32(0.5)
        for _ in range(3):
            y = y * (jnp.float32(1.5) - hx * y * y)
        out_v[sl] = dot * y
        return carry

    lax.fori_loop(0, NGRP, block_body, jnp.int32(0))

    pltpu.sync_copy(out_v, out_hbm.at[pl.ds(base, BPW)])


def kernel(e_id, o_id, mentees, mentors):
    tail_e = mentees[TAIL_START:, :].T
    tail_o = mentors[TAIL_START:, :].T
    return _cosine_sc(e_id, o_id, mentees.T, mentors.T, tail_e, tail_o)
